# baseline (speedup 1.0000x reference)
import functools

import jax
import jax.numpy as jnp
from jax import lax
from jax.experimental import pallas as pl
from jax.experimental.pallas import tpu as pltpu
from jax.experimental.pallas import tpu_sc as plsc

_NX = _NY = _NZ = 64
_PX, _PY, _PZ = _NX + 1, _NY + 1, _NZ + 1
_B = 128
_L = 16
_NW = 32
_GRP = _B // _L


def _sc_ffd(n_pad, blocks_per_worker):
    mesh = plsc.VectorSubcoreMesh(core_axis_name="c", subcore_axis_name="s")
    fdt = jnp.float32
    idt = jnp.int32
    nblk = n_pad // _B
    out_t = jax.ShapeDtypeStruct((nblk, 3 * _B), fdt)
    one_set = (
        [pltpu.VMEM((3 * _B,), fdt)]
        + [pltpu.VMEM((_B,), idt) for _ in range(8)]
        + [pltpu.VMEM((_B, _L), idt) for _ in range(8)]
        + [pltpu.VMEM((3 * _B,), fdt)]
        + [pltpu.SemaphoreType.DMA, pltpu.SemaphoreType.DMA]
    )
    scratch = [pltpu.VMEM((96,), fdt)] + one_set + one_set

    @functools.partial(
        pl.kernel, mesh=mesh, out_type=out_t, scratch_types=scratch,
        compiler_params=pltpu.CompilerParams(
            needs_layout_passes=False, use_tc_tiling_on_sc=False),
    )
    def k(e_hbm, v_hbm, par_hbm, o_hbm, *refs):
        pbuf = refs[0]
        sets = []
        for i in range(2):
            s0 = 1 + i * 20
            sets.append(dict(
                vb=refs[s0],
                idxb=refs[s0 + 1:s0 + 9],
                rowb=refs[s0 + 9:s0 + 17],
                ob=refs[s0 + 17],
                sem=refs[s0 + 18],
                osem=refs[s0 + 19],
            ))

        wid = lax.axis_index("s") * 2 + lax.axis_index("c")
        pltpu.sync_copy(par_hbm, pbuf)
        lanes = lax.iota(idt, _L)

        def par(i):
            return pbuf[pl.ds(16 * i, _L)]

        def cell(p, o, s):
            rel = (p - o) * s
            b = rel.astype(idt)
            b = jnp.minimum(jnp.maximum(b, 0), 61)
            return b, rel - b.astype(fdt)

        def basis(u):
            u2 = u * u
            u3 = u2 * u
            return (
                (1.0 - 3.0 * u + 3.0 * u2 - u3) * (1.0 / 6.0),
                (4.0 - 6.0 * u2 + 3.0 * u3) * (1.0 / 6.0),
                (1.0 + 3.0 * u + 3.0 * u2 - 3.0 * u3) * (1.0 / 6.0),
                u3 * (1.0 / 6.0),
            )

        def stage(blk, s):
            gb = wid * blocks_per_worker + blk
            pltpu.sync_copy(v_hbm.at[gb], s["vb"])

            def idx_grp(g, _):
                sl = pl.ds(g * _L, _L)
                bx, _u = cell(s["vb"][pl.ds(g * _L, _L)], par(0), par(3))
                by, _u = cell(s["vb"][pl.ds(_B + g * _L, _L)], par(1), par(4))
                bz, _u = cell(s["vb"][pl.ds(2 * _B + g * _L, _L)],
                              par(2), par(5))
                t = (bx * _PY + by) * _PZ + bz
                for a in range(4):
                    for bp in range(2):
                        s["idxb"][a * 2 + bp][sl] = (
                            t + (a * _PY + 2 * bp) * _PZ)
                return _
            lax.fori_loop(0, _GRP, idx_grp, None)
            for i in range(8):
                pltpu.async_copy(e_hbm.at[s["idxb"][i]], s["rowb"][i],
                                 s["sem"])

        def compute(blk, s, has_out):
            gb = wid * blocks_per_worker + blk
            for i in range(8):
                pltpu.make_async_copy(e_hbm.at[s["idxb"][i]], s["rowb"][i],
                                      s["sem"]).wait()
            if has_out:
                pltpu.make_async_copy(s["ob"], o_hbm.at[gb],
                                      s["osem"]).wait()

            def fma_grp(g, _):
                px = s["vb"][pl.ds(g * _L, _L)]
                py = s["vb"][pl.ds(_B + g * _L, _L)]
                pz = s["vb"][pl.ds(2 * _B + g * _L, _L)]
                _b, ux = cell(px, par(0), par(3))
                _b, uy = cell(py, par(1), par(4))
                _b, uz = cell(pz, par(2), par(5))
                bu = basis(ux)
                bv = basis(uy)
                bw = basis(uz)
                jv = lanes + g * _L
                acc = [px, py, pz]
                mask_hi = jnp.full((_L,), -65536, idt)
                sh16 = jnp.full((_L,), 16, idt)
                wabc = {}
                for a in range(4):
                    for b in range(4):
                        wab = bu[a] * bv[b]
                        for c in range(4):
                            wabc[(a, b, c)] = wab * bw[c]
                for a in range(4):
                    for bp in range(2):
                        r = s["rowb"][a * 2 + bp]
                        for kk in range(12):
                            kv = jnp.full((_L,), kk, idt)
                            g32 = plsc.load_gather(r, [jv, kv])
                            for h in range(2):
                                j = 2 * kk + h
                                yy, rr = j // 12, j % 12
                                c, d = rr // 3, rr % 3
                                if h == 0:
                                    bits = lax.shift_left(g32, sh16)
                                else:
                                    bits = lax.bitwise_and(g32, mask_hi)
                                val = lax.bitcast_convert_type(bits, fdt)
                                w = wabc[(a, 2 * bp + yy, c)]
                                acc[d] = acc[d] + w * val
                for d in range(3):
                    s["ob"][pl.ds(d * _B + g * _L, _L)] = acc[d]
                return _
            lax.fori_loop(0, _GRP, fma_grp, None)
            pltpu.async_copy(s["ob"], o_hbm.at[gb], s["osem"])

        stage(0, sets[0])
        stage(1, sets[1])
        compute(0, sets[0], has_out=False)
        stage(2, sets[0])
        compute(1, sets[1], has_out=False)

        def body(i, _):
            b0 = 2 * i
            stage(b0 + 1, sets[1])
            compute(b0, sets[0], has_out=True)

            @pl.when(b0 + 2 < blocks_per_worker)
            def _fire_next():
                stage(b0 + 2, sets[0])

            compute(b0 + 1, sets[1], has_out=True)
            return _

        lax.fori_loop(1, blocks_per_worker // 2, body, None)
        for s in sets:
            pltpu.make_async_copy(s["ob"], o_hbm.at[0], s["osem"]).wait()

    return k


def kernel(verts, deltaG, origin, spacing):
    n = verts.shape[0]
    step = 2 * _NW * _B
    n_pad = max(((n + step - 1) // step) * step, 2 * step)
    bpw = n_pad // (_NW * _B)

    pad = jnp.pad(deltaG.astype(jnp.bfloat16),
                  ((1, 0), (1, 1), (1, 3), (0, 0)))
    e24 = jnp.concatenate(
        [pad[:, yy:yy + _PY, c:c + _PZ, :] for yy in range(2)
         for c in range(4)], axis=-1)
    e = jnp.pad(e24, ((0, 0), (0, 0), (0, 0), (0, 8)))
    e = e.reshape(_PX * _PY * _PZ, 16, 2)
    e = lax.bitcast_convert_type(e, jnp.int32)

    nblk = n_pad // _B
    vpack = jnp.pad(verts.astype(jnp.float32),
                    ((0, n_pad - n), (0, 0)))
    vpack = vpack.reshape(nblk, _B, 3).transpose(0, 2, 1).reshape(nblk, 3 * _B)

    inv_sp = 1.0 / spacing.astype(jnp.float32)
    par = jnp.concatenate([
        jnp.broadcast_to(origin.astype(jnp.float32)[i], (16,))
        for i in range(3)
    ] + [jnp.broadcast_to(inv_sp[i], (16,)) for i in range(3)])

    o = _sc_ffd(n_pad, bpw)(e, vpack, par)
    o = o.reshape(nblk, 3, _B).transpose(0, 2, 1).reshape(n_pad, 3)
    return o[:n]


# speedup vs baseline: 1.4645x; 1.4645x over previous
import functools

import jax
import jax.numpy as jnp
from jax import lax
from jax.experimental import pallas as pl
from jax.experimental.pallas import tpu as pltpu
from jax.experimental.pallas import tpu_sc as plsc

_NX = _NY = _NZ = 64
_PX, _PY, _PZ = _NX + 1, _NY + 1, _NZ + 1
_B = 128
_L = 16
_NW = 32
_GRP = _B // _L


def _sc_ffd(n_pad, bpw):
    mesh = plsc.VectorSubcoreMesh(core_axis_name="c", subcore_axis_name="s")
    fdt = jnp.float32
    idt = jnp.int32
    nblk = n_pad // _B
    out_t = jax.ShapeDtypeStruct((nblk, 3 * _B), fdt)
    one_set = (
        [pltpu.VMEM((_B,), idt) for _ in range(16)]
        + [pltpu.VMEM((_B, _L), fdt) for _ in range(16)]
        + [pltpu.VMEM((3 * _B,), fdt)]
        + [pltpu.SemaphoreType.DMA, pltpu.SemaphoreType.DMA]
    )
    scratch = (
        [pltpu.VMEM((96,), fdt)]
        + [pltpu.VMEM((3 * _B,), fdt)] * 4
        + [pltpu.SemaphoreType.DMA] * 4
        + one_set + one_set
    )

    @functools.partial(
        pl.kernel, mesh=mesh, out_type=out_t, scratch_types=scratch,
        compiler_params=pltpu.CompilerParams(
            needs_layout_passes=False, use_tc_tiling_on_sc=False),
    )
    def k(e_hbm, v_hbm, par_hbm, o_hbm, *refs):
        pbuf = refs[0]
        vring = refs[1:5]
        vsems = refs[5:9]
        sets = []
        for i in range(2):
            s0 = 9 + i * 35
            sets.append(dict(
                idxb=refs[s0:s0 + 16],
                rowb=refs[s0 + 16:s0 + 32],
                ob=refs[s0 + 32],
                sem=refs[s0 + 33],
                osem=refs[s0 + 34],
            ))

        wid = lax.axis_index("s") * 2 + lax.axis_index("c")
        pltpu.sync_copy(par_hbm, pbuf)
        lanes = lax.iota(idt, _L)

        def par(i):
            return pbuf[pl.ds(16 * i, _L)]

        def cell(p, o, s):
            rel = (p - o) * s
            b = rel.astype(idt)
            b = jnp.minimum(jnp.maximum(b, 0), 61)
            return b, rel - b.astype(fdt)

        def basis(u):
            u2 = u * u
            u3 = u2 * u
            return (
                (1.0 - 3.0 * u + 3.0 * u2 - u3) * (1.0 / 6.0),
                (4.0 - 6.0 * u2 + 3.0 * u3) * (1.0 / 6.0),
                (1.0 + 3.0 * u + 3.0 * u2 - 3.0 * u3) * (1.0 / 6.0),
                u3 * (1.0 / 6.0),
            )

        def vload(blk, slot, guard):
            def fire():
                pltpu.async_copy(v_hbm.at[wid * bpw + blk], vring[slot],
                                 vsems[slot])
            if guard:
                pl.when(blk < bpw)(fire)
            else:
                fire()

        def stage(blk, s, slot):
            pltpu.make_async_copy(v_hbm.at[wid * bpw + blk], vring[slot],
                                  vsems[slot]).wait()
            vb = vring[slot]

            def idx_grp(g, _):
                sl = pl.ds(g * _L, _L)
                bx, _u = cell(vb[pl.ds(g * _L, _L)], par(0), par(3))
                by, _u = cell(vb[pl.ds(_B + g * _L, _L)], par(1), par(4))
                bz, _u = cell(vb[pl.ds(2 * _B + g * _L, _L)], par(2), par(5))
                t = (bx * _PY + by) * _PZ + bz
                for a in range(4):
                    for b in range(4):
                        s["idxb"][a * 4 + b][sl] = t + (a * _PY + b) * _PZ
                return _
            lax.fori_loop(0, _GRP, idx_grp, None)
            for i in range(16):
                pltpu.async_copy(e_hbm.at[s["idxb"][i]], s["rowb"][i],
                                 s["sem"])

        def compute(blk, s, slot, has_out):
            gb = wid * bpw + blk
            for i in range(16):
                pltpu.make_async_copy(e_hbm.at[s["idxb"][i]], s["rowb"][i],
                                      s["sem"]).wait()
            if has_out:
                pltpu.make_async_copy(s["ob"], o_hbm.at[gb],
                                      s["osem"]).wait()
            vb = vring[slot]

            def fma_grp(g, _):
                px = vb[pl.ds(g * _L, _L)]
                py = vb[pl.ds(_B + g * _L, _L)]
                pz = vb[pl.ds(2 * _B + g * _L, _L)]
                _b, ux = cell(px, par(0), par(3))
                _b, uy = cell(py, par(1), par(4))
                _b, uz = cell(pz, par(2), par(5))
                bu = basis(ux)
                bv = basis(uy)
                bw = basis(uz)
                jv = lanes + g * _L
                acc = [px, py, pz]
                for a in range(4):
                    for b in range(4):
                        wab = bu[a] * bv[b]
                        r = s["rowb"][a * 4 + b]
                        for c in range(4):
                            wabc = wab * bw[c]
                            for d in range(3):
                                kv = jnp.full((_L,), 3 * c + d, idt)
                                g16 = plsc.load_gather(r, [jv, kv])
                                acc[d] = acc[d] + wabc * g16
                for d in range(3):
                    s["ob"][pl.ds(d * _B + g * _L, _L)] = acc[d]
                return _
            lax.fori_loop(0, _GRP, fma_grp, None)
            pltpu.async_copy(s["ob"], o_hbm.at[gb], s["osem"])

        for b in range(4):
            vload(b, b, guard=False)
        stage(0, sets[0], 0)
        stage(1, sets[1], 1)
        compute(0, sets[0], 0, has_out=False)
        vload(4, 0, guard=False)
        stage(2, sets[0], 2)
        compute(1, sets[1], 1, has_out=False)
        vload(5, 1, guard=False)
        stage(3, sets[1], 3)

        def body(i, _):
            b = 4 * i + 2
            for j in range(4):
                blk = b + j
                st = sets[j % 2]
                compute(blk, st, (2 + j) % 4, has_out=True)
                vload(blk + 4, (2 + j) % 4, guard=True)
                stage(blk + 2, st, j % 4)
            return _

        lax.fori_loop(0, (bpw - 4) // 4, body, None)

        compute(bpw - 2, sets[0], (bpw - 2) % 4, has_out=True)
        compute(bpw - 1, sets[1], (bpw - 1) % 4, has_out=True)
        for s in sets:
            pltpu.make_async_copy(s["ob"], o_hbm.at[0], s["osem"]).wait()

    return k


def kernel(verts, deltaG, origin, spacing):
    n = verts.shape[0]
    step = 4 * _NW * _B
    n_pad = max(((n + step - 1) // step) * step, 2 * step)
    bpw = n_pad // (_NW * _B)

    pad = jnp.pad(deltaG.astype(jnp.float32),
                  ((1, 0), (1, 0), (1, 3), (0, 0)))
    e12 = jnp.concatenate([pad[:, :, c:c + _PZ, :] for c in range(4)],
                          axis=-1)
    e = jnp.pad(e12, ((0, 0), (0, 0), (0, 0), (0, 4)))
    e = e.reshape(_PX * _PY * _PZ, 16)

    nblk = n_pad // _B
    vpack = jnp.pad(verts.astype(jnp.float32),
                    ((0, n_pad - n), (0, 0)))
    vpack = vpack.reshape(nblk, _B, 3).transpose(0, 2, 1).reshape(nblk, 3 * _B)

    inv_sp = 1.0 / spacing.astype(jnp.float32)
    par = jnp.concatenate([
        jnp.broadcast_to(origin.astype(jnp.float32)[i], (16,))
        for i in range(3)
    ] + [jnp.broadcast_to(inv_sp[i], (16,)) for i in range(3)])

    o = _sc_ffd(n_pad, bpw)(e, vpack, par)
    o = o.reshape(nblk, 3, _B).transpose(0, 2, 1).reshape(n_pad, 3)
    return o[:n]
